# baseline (speedup 1.0000x reference)
import functools

import jax
import jax.numpy as jnp
from jax import lax
from jax.experimental import pallas as pl
from jax.experimental.pallas import tpu as pltpu
from jax.experimental.pallas import tpu_sc as plsc

S = 2048
D = 768
FF = 3072
E = 8
CAP = 320
SLOT_PER_E = 328
NSLOT = E * SLOT_PER_E
NROW = NSLOT + S

NC = 2
NS = 16
NW = NC * NS
TOK_PER_W = S // NW


def _router_body(hs_ref, wcls_ref, logits_ref, slot_ref, fei_ref, xs_ref,
                 prow_ref):
    logits = jnp.dot(hs_ref[...], wcls_ref[...],
                     preferred_element_type=jnp.float32)
    logits_ref[...] = logits
    lmax = jnp.max(logits, axis=1, keepdims=True)
    p = 1.0 / jnp.sum(jnp.exp(logits - lmax), axis=1, keepdims=True)
    col = lax.broadcasted_iota(jnp.int32, (S, E), 1)
    eidx = jnp.min(jnp.where(logits == lmax, col, E), axis=1,
                   keepdims=True)
    onehot = (col == eidx).astype(jnp.float32)
    cum = onehot
    k = 1
    while k < S:
        cum = cum + jnp.concatenate(
            [jnp.zeros((k, E), jnp.float32), cum[:S - k, :]], axis=0)
        k *= 2
    prio = jnp.sum(cum * onehot, axis=1, keepdims=True)
    routed = prio <= float(CAP)
    rank = (prio - 1.0).astype(jnp.int32)
    row = lax.broadcasted_iota(jnp.int32, (S, 1), 0)
    slot_ref[...] = jnp.where(routed, eidx * SLOT_PER_E + rank, NSLOT + row)
    fei_ref[...] = jnp.where(routed, eidx, 0)
    xs_ref[...] = jnp.where(routed, hs_ref[...], p * hs_ref[...])
    prow_ref[...] = jnp.broadcast_to(p, (S, 128))


def _router_call(hs, wcls):
    return pl.pallas_call(
        _router_body,
        out_shape=(
            jax.ShapeDtypeStruct((S, E), jnp.float32),
            jax.ShapeDtypeStruct((S, 1), jnp.int32),
            jax.ShapeDtypeStruct((S, 1), jnp.int32),
            jax.ShapeDtypeStruct((S, D), jnp.float32),
            jax.ShapeDtypeStruct((S, 128), jnp.float32),
        ),
    )(hs, wcls)


@functools.cache
def _make_dispatch():
    mesh = plsc.VectorSubcoreMesh(core_axis_name="c", subcore_axis_name="s")

    @functools.partial(
        pl.kernel, mesh=mesh,
        out_type=(
            jax.ShapeDtypeStruct((NROW, D), jnp.float32),
            jax.ShapeDtypeStruct((NROW, 128), jnp.float32),
        ),
        scratch_types=[
            pltpu.VMEM((TOK_PER_W,), jnp.int32),
            pltpu.VMEM((TOK_PER_W, D), jnp.float32),
            pltpu.VMEM((TOK_PER_W, 128), jnp.float32),
            pltpu.SemaphoreType.DMA,
        ],
    )
    def _dispatch(xs_hbm, prow_hbm, slot_hbm, xbuf_hbm, pbuf_hbm,
                  idx_v, rows_v, p_v, sem):
        wid = lax.axis_index("s") * NC + lax.axis_index("c")
        base = wid * TOK_PER_W
        pltpu.sync_copy(slot_hbm.at[pl.ds(base, TOK_PER_W)], idx_v)
        pltpu.sync_copy(xs_hbm.at[pl.ds(base, TOK_PER_W)], rows_v)
        pltpu.sync_copy(prow_hbm.at[pl.ds(base, TOK_PER_W)], p_v)
        pltpu.async_copy(rows_v, xbuf_hbm.at[idx_v], sem).wait()
        pltpu.async_copy(p_v, pbuf_hbm.at[idx_v], sem).wait()

    return _dispatch


HF = FF // 2


def _ffn_body(x_ref, p_ref, wia_ref, wib_ref, woa_ref, wob_ref, y_ref):
    x = x_ref[...]
    ha = jnp.maximum(
        jnp.dot(x, wia_ref[0], preferred_element_type=jnp.float32), 0.0)
    hb = jnp.maximum(
        jnp.dot(x, wib_ref[0], preferred_element_type=jnp.float32), 0.0)
    y = (jnp.dot(ha, woa_ref[0], preferred_element_type=jnp.float32)
         + jnp.dot(hb, wob_ref[0], preferred_element_type=jnp.float32))
    y_ref[...] = p_ref[...][:, 0:1] * y


def _ffn_call(xbuf, pbuf, wi, wo):
    return pl.pallas_call(
        _ffn_body,
        grid=(E,),
        in_specs=[
            pl.BlockSpec((SLOT_PER_E, D), lambda e: (e, 0)),
            pl.BlockSpec((SLOT_PER_E, 128), lambda e: (e, 0)),
            pl.BlockSpec((1, D, HF), lambda e: (e, 0, 0)),
            pl.BlockSpec((1, D, HF), lambda e: (e, 0, 1)),
            pl.BlockSpec((1, HF, D), lambda e: (e, 0, 0)),
            pl.BlockSpec((1, HF, D), lambda e: (e, 1, 0)),
        ],
        out_specs=pl.BlockSpec((SLOT_PER_E, D), lambda e: (e, 0)),
        out_shape=jax.ShapeDtypeStruct((NROW, D), jnp.float32),
        input_output_aliases={0: 0},
    )(xbuf, pbuf, wi, wi, wo, wo)


@functools.cache
def _make_collect():
    mesh = plsc.VectorSubcoreMesh(core_axis_name="c", subcore_axis_name="s")

    @functools.partial(
        pl.kernel, mesh=mesh,
        out_type=jax.ShapeDtypeStruct((S, D), jnp.float32),
        scratch_types=[
            pltpu.VMEM((TOK_PER_W,), jnp.int32),
            pltpu.VMEM((TOK_PER_W, D), jnp.float32),
            pltpu.SemaphoreType.DMA,
        ],
    )
    def _collect(y_hbm, slot_hbm, out_hbm, idx_v, rows_v, sem):
        wid = lax.axis_index("s") * NC + lax.axis_index("c")
        base = wid * TOK_PER_W
        pltpu.sync_copy(slot_hbm.at[pl.ds(base, TOK_PER_W)], idx_v)
        pltpu.async_copy(y_hbm.at[idx_v], rows_v, sem).wait()
        pltpu.sync_copy(rows_v, out_hbm.at[pl.ds(base, TOK_PER_W)])

    return _collect


def kernel(hidden_states, W_cls, wi, wo):
    hs = hidden_states.reshape(S, D)
    logits, slot, fei, xs, prow = _router_call(hs, W_cls)
    slot_flat = slot.reshape(S)
    xbuf, pbuf = _make_dispatch()(xs, prow, slot_flat)
    y = _ffn_call(xbuf, pbuf, wi, wo)
    out = _make_collect()(y, slot_flat)
    return (out.reshape(1, S, D),
            logits.reshape(1, S, E),
            fei.reshape(1, S))


# speedup vs baseline: 1.0035x; 1.0035x over previous
import functools

import jax
import jax.numpy as jnp
from jax import lax
from jax.experimental import pallas as pl
from jax.experimental.pallas import tpu as pltpu
from jax.experimental.pallas import tpu_sc as plsc

S = 2048
D = 768
FF = 3072
E = 8
CAP = 320
SLOT_PER_E = 320
NSLOT = E * SLOT_PER_E
NROW = NSLOT + S

NC = 2
NS = 16
NW = NC * NS
TOK_PER_W = S // NW


def _router_body(hs_ref, wcls_ref, logits_ref, slot_ref, fei_ref, xs_ref,
                 prow_ref):
    logits = jnp.dot(hs_ref[...], wcls_ref[...],
                     preferred_element_type=jnp.float32)
    logits_ref[...] = logits
    lmax = jnp.max(logits, axis=1, keepdims=True)
    p = 1.0 / jnp.sum(jnp.exp(logits - lmax), axis=1, keepdims=True)
    col = lax.broadcasted_iota(jnp.int32, (S, E), 1)
    eidx = jnp.min(jnp.where(logits == lmax, col, E), axis=1,
                   keepdims=True)
    onehot = (col == eidx).astype(jnp.float32)
    cum = onehot
    k = 1
    while k < S:
        cum = cum + jnp.concatenate(
            [jnp.zeros((k, E), jnp.float32), cum[:S - k, :]], axis=0)
        k *= 2
    prio = jnp.sum(cum * onehot, axis=1, keepdims=True)
    routed = prio <= float(CAP)
    rank = (prio - 1.0).astype(jnp.int32)
    row = lax.broadcasted_iota(jnp.int32, (S, 1), 0)
    slot_ref[...] = jnp.where(routed, eidx * SLOT_PER_E + rank, NSLOT + row)
    fei_ref[...] = jnp.where(routed, eidx, 0)
    xs_ref[...] = jnp.where(routed, hs_ref[...], p * hs_ref[...])
    prow_ref[...] = jnp.broadcast_to(p, (S, 128))


def _router_call(hs, wcls):
    return pl.pallas_call(
        _router_body,
        out_shape=(
            jax.ShapeDtypeStruct((S, E), jnp.float32),
            jax.ShapeDtypeStruct((S, 1), jnp.int32),
            jax.ShapeDtypeStruct((S, 1), jnp.int32),
            jax.ShapeDtypeStruct((S, D), jnp.float32),
            jax.ShapeDtypeStruct((S, 128), jnp.float32),
        ),
    )(hs, wcls)


@functools.cache
def _make_dispatch():
    mesh = plsc.VectorSubcoreMesh(core_axis_name="c", subcore_axis_name="s")

    @functools.partial(
        pl.kernel, mesh=mesh,
        out_type=(
            jax.ShapeDtypeStruct((NROW, D), jnp.float32),
            jax.ShapeDtypeStruct((NROW, 128), jnp.float32),
        ),
        scratch_types=[
            pltpu.VMEM((TOK_PER_W,), jnp.int32),
            pltpu.VMEM((TOK_PER_W, D), jnp.float32),
            pltpu.VMEM((TOK_PER_W, 128), jnp.float32),
            pltpu.SemaphoreType.DMA,
        ],
    )
    def _dispatch(xs_hbm, prow_hbm, slot_hbm, xbuf_hbm, pbuf_hbm,
                  idx_v, rows_v, p_v, sem):
        wid = lax.axis_index("s") * NC + lax.axis_index("c")
        base = wid * TOK_PER_W
        pltpu.sync_copy(slot_hbm.at[pl.ds(base, TOK_PER_W)], idx_v)
        pltpu.sync_copy(xs_hbm.at[pl.ds(base, TOK_PER_W)], rows_v)
        pltpu.sync_copy(prow_hbm.at[pl.ds(base, TOK_PER_W)], p_v)
        pltpu.async_copy(rows_v, xbuf_hbm.at[idx_v], sem).wait()
        pltpu.async_copy(p_v, pbuf_hbm.at[idx_v], sem).wait()

    return _dispatch


def _ffn_body(x_ref, p_ref, wi_ref, wo_ref, y_ref):
    h = jnp.maximum(
        jnp.dot(x_ref[...], wi_ref[0], preferred_element_type=jnp.float32),
        0.0)
    y = jnp.dot(h, wo_ref[0], preferred_element_type=jnp.float32)
    y_ref[...] = p_ref[...][:, 0:1] * y


def _ffn_call(xbuf, pbuf, wi, wo):
    return pl.pallas_call(
        _ffn_body,
        grid=(E,),
        in_specs=[
            pl.BlockSpec((SLOT_PER_E, D), lambda e: (e, 0)),
            pl.BlockSpec((SLOT_PER_E, 128), lambda e: (e, 0)),
            pl.BlockSpec((1, D, FF), lambda e: (e, 0, 0)),
            pl.BlockSpec((1, FF, D), lambda e: (e, 0, 0)),
        ],
        out_specs=pl.BlockSpec((SLOT_PER_E, D), lambda e: (e, 0)),
        out_shape=jax.ShapeDtypeStruct((NROW, D), jnp.float32),
        input_output_aliases={0: 0},
    )(xbuf, pbuf, wi, wo)


@functools.cache
def _make_collect():
    mesh = plsc.VectorSubcoreMesh(core_axis_name="c", subcore_axis_name="s")

    @functools.partial(
        pl.kernel, mesh=mesh,
        out_type=jax.ShapeDtypeStruct((S, D), jnp.float32),
        scratch_types=[
            pltpu.VMEM((TOK_PER_W,), jnp.int32),
            pltpu.VMEM((TOK_PER_W, D), jnp.float32),
            pltpu.SemaphoreType.DMA,
        ],
    )
    def _collect(y_hbm, slot_hbm, out_hbm, idx_v, rows_v, sem):
        wid = lax.axis_index("s") * NC + lax.axis_index("c")
        base = wid * TOK_PER_W
        pltpu.sync_copy(slot_hbm.at[pl.ds(base, TOK_PER_W)], idx_v)
        pltpu.async_copy(y_hbm.at[idx_v], rows_v, sem).wait()
        pltpu.sync_copy(rows_v, out_hbm.at[pl.ds(base, TOK_PER_W)])

    return _collect


def kernel(hidden_states, W_cls, wi, wo):
    hs = hidden_states.reshape(S, D)
    logits, slot, fei, xs, prow = _router_call(hs, W_cls)
    slot_flat = slot.reshape(S)
    xbuf, pbuf = _make_dispatch()(xs, prow, slot_flat)
    y = _ffn_call(xbuf, pbuf, wi, wo)
    out = _make_collect()(y, slot_flat)
    return (out.reshape(1, S, D),
            logits.reshape(1, S, E),
            fei.reshape(1, S))


# speedup vs baseline: 1.0288x; 1.0252x over previous
import functools

import jax
import jax.numpy as jnp
from jax import lax
from jax.experimental import pallas as pl
from jax.experimental.pallas import tpu as pltpu
from jax.experimental.pallas import tpu_sc as plsc

S = 2048
D = 768
FF = 3072
E = 8
CAP = 320
SLOT_PER_E = 320
NSLOT = E * SLOT_PER_E
NROW = NSLOT + S

NC = 2
NS = 16
NW = NC * NS
TOK_PER_W = S // NW
L = 16


def _router_body(hs_ref, wcls_ref, logits_ref, packed_ref, xs_ref, prow_ref):
    logits = jnp.dot(hs_ref[...], wcls_ref[...],
                     preferred_element_type=jnp.float32)
    logits_ref[...] = logits
    lmax = jnp.max(logits, axis=1, keepdims=True)
    p = 1.0 / jnp.sum(jnp.exp(logits - lmax), axis=1, keepdims=True)
    col = lax.broadcasted_iota(jnp.int32, (S, E), 1)
    eidx = jnp.min(jnp.where(logits == lmax, col, E), axis=1,
                   keepdims=True)
    onehot = (col == eidx).astype(jnp.float32)
    cum = onehot
    k = 1
    while k < S:
        cum = cum + jnp.concatenate(
            [jnp.zeros((k, E), jnp.float32), cum[:S - k, :]], axis=0)
        k *= 2
    prio = jnp.sum(cum * onehot, axis=1, keepdims=True)
    routed = prio <= float(CAP)
    rank = (prio - 1.0).astype(jnp.int32)
    row = lax.broadcasted_iota(jnp.int32, (S, 1), 0)
    slot = jnp.where(routed, eidx * SLOT_PER_E + rank, NSLOT + row)
    fei = jnp.where(routed, eidx, 0)
    packed_ref[...] = jnp.broadcast_to(slot | (fei << 16), (S, 128))
    xs_ref[...] = jnp.where(routed, hs_ref[...], p * hs_ref[...])
    prow_ref[...] = jnp.broadcast_to(p, (S, 128))


def _router_call(hs, wcls):
    return pl.pallas_call(
        _router_body,
        out_shape=(
            jax.ShapeDtypeStruct((S, E), jnp.float32),
            jax.ShapeDtypeStruct((S, 128), jnp.int32),
            jax.ShapeDtypeStruct((S, D), jnp.float32),
            jax.ShapeDtypeStruct((S, 128), jnp.float32),
        ),
    )(hs, wcls)


def _extract_column0(pk_v, dst_v, shift, mask):
    lane = lax.iota(jnp.int32, L)
    for g in range(TOK_PER_W // L):
        acc = jnp.zeros((L,), jnp.int32)
        for k in range(L):
            v = pk_v[g * L + k, pl.ds(0, L)]
            acc = jnp.where(lane == k, v, acc)
        if shift:
            acc = lax.shift_right_logical(acc, shift)
        dst_v[pl.ds(L * g, L)] = lax.bitwise_and(acc, mask)


@functools.cache
def _make_dispatch():
    mesh = plsc.VectorSubcoreMesh(core_axis_name="c", subcore_axis_name="s")

    @functools.partial(
        pl.kernel, mesh=mesh,
        out_type=(
            jax.ShapeDtypeStruct((NROW, D), jnp.float32),
            jax.ShapeDtypeStruct((NROW, 128), jnp.float32),
        ),
        scratch_types=[
            pltpu.VMEM((TOK_PER_W, 128), jnp.int32),
            pltpu.VMEM((TOK_PER_W,), jnp.int32),
            pltpu.VMEM((TOK_PER_W, D), jnp.float32),
            pltpu.VMEM((TOK_PER_W, 128), jnp.float32),
            pltpu.SemaphoreType.DMA,
        ],
    )
    def _dispatch(xs_hbm, prow_hbm, packed_hbm, xbuf_hbm, pbuf_hbm,
                  pk_v, idx_v, rows_v, p_v, sem):
        wid = lax.axis_index("s") * NC + lax.axis_index("c")
        base = wid * TOK_PER_W
        pltpu.sync_copy(packed_hbm.at[pl.ds(base, TOK_PER_W)], pk_v)
        pltpu.sync_copy(xs_hbm.at[pl.ds(base, TOK_PER_W)], rows_v)
        pltpu.sync_copy(prow_hbm.at[pl.ds(base, TOK_PER_W)], p_v)
        _extract_column0(pk_v, idx_v, 0, 0xFFFF)
        pltpu.async_copy(rows_v, xbuf_hbm.at[idx_v], sem).wait()
        pltpu.async_copy(p_v, pbuf_hbm.at[idx_v], sem).wait()

    return _dispatch


def _ffn_body(x_ref, p_ref, wi_ref, wo_ref, y_ref):
    h = jnp.maximum(
        jnp.dot(x_ref[...], wi_ref[0], preferred_element_type=jnp.float32),
        0.0)
    y = jnp.dot(h, wo_ref[0], preferred_element_type=jnp.float32)
    y_ref[...] = p_ref[...][:, 0:1] * y


def _ffn_call(xbuf, pbuf, wi, wo):
    return pl.pallas_call(
        _ffn_body,
        grid=(E,),
        in_specs=[
            pl.BlockSpec((SLOT_PER_E, D), lambda e: (e, 0)),
            pl.BlockSpec((SLOT_PER_E, 128), lambda e: (e, 0)),
            pl.BlockSpec((1, D, FF), lambda e: (e, 0, 0)),
            pl.BlockSpec((1, FF, D), lambda e: (e, 0, 0)),
        ],
        out_specs=pl.BlockSpec((SLOT_PER_E, D), lambda e: (e, 0)),
        out_shape=jax.ShapeDtypeStruct((NROW, D), jnp.float32),
        input_output_aliases={0: 0},
    )(xbuf, pbuf, wi, wo)


@functools.cache
def _make_collect():
    mesh = plsc.VectorSubcoreMesh(core_axis_name="c", subcore_axis_name="s")

    @functools.partial(
        pl.kernel, mesh=mesh,
        out_type=(
            jax.ShapeDtypeStruct((S, D), jnp.float32),
            jax.ShapeDtypeStruct((S,), jnp.int32),
        ),
        scratch_types=[
            pltpu.VMEM((TOK_PER_W, 128), jnp.int32),
            pltpu.VMEM((TOK_PER_W,), jnp.int32),
            pltpu.VMEM((TOK_PER_W,), jnp.int32),
            pltpu.VMEM((TOK_PER_W, D), jnp.float32),
            pltpu.SemaphoreType.DMA,
        ],
    )
    def _collect(y_hbm, packed_hbm, out_hbm, fei_hbm,
                 pk_v, idx_v, fei_v, rows_v, sem):
        wid = lax.axis_index("s") * NC + lax.axis_index("c")
        base = wid * TOK_PER_W
        pltpu.sync_copy(packed_hbm.at[pl.ds(base, TOK_PER_W)], pk_v)
        _extract_column0(pk_v, idx_v, 0, 0xFFFF)
        _extract_column0(pk_v, fei_v, 16, 0xFF)
        pltpu.async_copy(y_hbm.at[idx_v], rows_v, sem).wait()
        pltpu.sync_copy(rows_v, out_hbm.at[pl.ds(base, TOK_PER_W)])
        pltpu.sync_copy(fei_v, fei_hbm.at[pl.ds(base, TOK_PER_W)])

    return _collect


def kernel(hidden_states, W_cls, wi, wo):
    hs = hidden_states.reshape(S, D)
    logits, packed, xs, prow = _router_call(hs, W_cls)
    xbuf, pbuf = _make_dispatch()(xs, prow, packed)
    y = _ffn_call(xbuf, pbuf, wi, wo)
    out, fei = _make_collect()(y, packed)
    return (out.reshape(1, S, D),
            logits.reshape(1, S, E),
            fei.reshape(1, S))


# speedup vs baseline: 1.0441x; 1.0148x over previous
import functools

import jax
import jax.numpy as jnp
from jax import lax
from jax.experimental import pallas as pl
from jax.experimental.pallas import tpu as pltpu
from jax.experimental.pallas import tpu_sc as plsc

S = 2048
D = 768
FF = 3072
E = 8
CAP = 320
SLOT_PER_E = 320
NSLOT = E * SLOT_PER_E
NROW = NSLOT + S

NC = 2
NS = 16
NW = NC * NS
TOK_PER_W = S // NW
L = 16


def _router_body(hs_ref, wcls_ref, logits_ref, packed_ref, xs_ref, prow_ref):
    logits = jnp.dot(hs_ref[...], wcls_ref[...],
                     preferred_element_type=jnp.float32)
    logits_ref[...] = logits
    lmax = jnp.max(logits, axis=1, keepdims=True)
    p = 1.0 / jnp.sum(jnp.exp(logits - lmax), axis=1, keepdims=True)
    col = lax.broadcasted_iota(jnp.int32, (S, E), 1)
    eidx = jnp.min(jnp.where(logits == lmax, col, E), axis=1,
                   keepdims=True)
    onehot = (col == eidx).astype(jnp.float32)
    cum = onehot
    k = 1
    while k < S:
        cum = cum + jnp.concatenate(
            [jnp.zeros((k, E), jnp.float32), cum[:S - k, :]], axis=0)
        k *= 2
    prio = jnp.sum(cum * onehot, axis=1, keepdims=True)
    routed = prio <= float(CAP)
    rank = (prio - 1.0).astype(jnp.int32)
    row = lax.broadcasted_iota(jnp.int32, (S, 1), 0)
    slot = jnp.where(routed, eidx * SLOT_PER_E + rank, NSLOT + row)
    fei = jnp.where(routed, eidx, 0)
    packed_ref[...] = jnp.broadcast_to(slot | (fei << 16), (S, 128))
    xs_ref[...] = jnp.where(routed, hs_ref[...], p * hs_ref[...])
    prow_ref[...] = jnp.broadcast_to(p, (S, 128))


def _router_call(hs, wcls):
    return pl.pallas_call(
        _router_body,
        out_shape=(
            jax.ShapeDtypeStruct((S, E), jnp.float32),
            jax.ShapeDtypeStruct((S, 128), jnp.int32),
            jax.ShapeDtypeStruct((S, D), jnp.float32),
            jax.ShapeDtypeStruct((S, 128), jnp.float32),
        ),
    )(hs, wcls)


def _extract_column0(pk_v, dst_v, shift, mask):
    lane = lax.iota(jnp.int32, L)
    for g in range(TOK_PER_W // L):
        acc = jnp.zeros((L,), jnp.int32)
        for k in range(L):
            v = pk_v[g * L + k, pl.ds(0, L)]
            acc = jnp.where(lane == k, v, acc)
        if shift:
            acc = lax.shift_right_logical(acc, shift)
        dst_v[pl.ds(L * g, L)] = lax.bitwise_and(acc, mask)


@functools.cache
def _make_dispatch():
    mesh = plsc.VectorSubcoreMesh(core_axis_name="c", subcore_axis_name="s")

    @functools.partial(
        pl.kernel, mesh=mesh,
        out_type=(
            jax.ShapeDtypeStruct((NROW, D), jnp.float32),
            jax.ShapeDtypeStruct((NROW, 128), jnp.float32),
        ),
        scratch_types=[
            pltpu.VMEM((TOK_PER_W, 128), jnp.int32),
            pltpu.VMEM((TOK_PER_W,), jnp.int32),
            pltpu.VMEM((TOK_PER_W, D), jnp.float32),
            pltpu.VMEM((TOK_PER_W, 128), jnp.float32),
            pltpu.SemaphoreType.DMA,
            pltpu.SemaphoreType.DMA,
            pltpu.SemaphoreType.DMA,
        ],
    )
    def _dispatch(xs_hbm, prow_hbm, packed_hbm, xbuf_hbm, pbuf_hbm,
                  pk_v, idx_v, rows_v, p_v, sem_x, sem_p, sem_s):
        wid = lax.axis_index("s") * NC + lax.axis_index("c")
        base = wid * TOK_PER_W
        cx = pltpu.async_copy(xs_hbm.at[pl.ds(base, TOK_PER_W)], rows_v, sem_x)
        cp = pltpu.async_copy(prow_hbm.at[pl.ds(base, TOK_PER_W)], p_v, sem_p)
        pltpu.sync_copy(packed_hbm.at[pl.ds(base, TOK_PER_W)], pk_v)
        _extract_column0(pk_v, idx_v, 0, 0xFFFF)
        cp.wait()
        sp = pltpu.async_copy(p_v, pbuf_hbm.at[idx_v], sem_s)
        cx.wait()
        pltpu.async_copy(rows_v, xbuf_hbm.at[idx_v], sem_x).wait()
        sp.wait()

    return _dispatch


def _ffn_body(x_ref, p_ref, wi_ref, wo_ref, y_ref):
    h = jnp.maximum(
        jnp.dot(x_ref[...], wi_ref[0], preferred_element_type=jnp.float32),
        0.0)
    y = jnp.dot(h, wo_ref[0], preferred_element_type=jnp.float32)
    y_ref[...] = p_ref[...][:, 0:1] * y


def _ffn_call(xbuf, pbuf, wi, wo):
    return pl.pallas_call(
        _ffn_body,
        grid=(E,),
        in_specs=[
            pl.BlockSpec((SLOT_PER_E, D), lambda e: (e, 0)),
            pl.BlockSpec((SLOT_PER_E, 128), lambda e: (e, 0)),
            pl.BlockSpec((1, D, FF), lambda e: (e, 0, 0)),
            pl.BlockSpec((1, FF, D), lambda e: (e, 0, 0)),
        ],
        out_specs=pl.BlockSpec((SLOT_PER_E, D), lambda e: (e, 0)),
        out_shape=jax.ShapeDtypeStruct((NROW, D), jnp.float32),
        input_output_aliases={0: 0},
    )(xbuf, pbuf, wi, wo)


@functools.cache
def _make_collect():
    mesh = plsc.VectorSubcoreMesh(core_axis_name="c", subcore_axis_name="s")

    @functools.partial(
        pl.kernel, mesh=mesh,
        out_type=(
            jax.ShapeDtypeStruct((S, D), jnp.float32),
            jax.ShapeDtypeStruct((S,), jnp.int32),
        ),
        scratch_types=[
            pltpu.VMEM((TOK_PER_W, 128), jnp.int32),
            pltpu.VMEM((TOK_PER_W,), jnp.int32),
            pltpu.VMEM((TOK_PER_W,), jnp.int32),
            pltpu.VMEM((TOK_PER_W, D), jnp.float32),
            pltpu.SemaphoreType.DMA,
        ],
    )
    def _collect(y_hbm, packed_hbm, out_hbm, fei_hbm,
                 pk_v, idx_v, fei_v, rows_v, sem):
        wid = lax.axis_index("s") * NC + lax.axis_index("c")
        base = wid * TOK_PER_W
        pltpu.sync_copy(packed_hbm.at[pl.ds(base, TOK_PER_W)], pk_v)
        _extract_column0(pk_v, idx_v, 0, 0xFFFF)
        cg = pltpu.async_copy(y_hbm.at[idx_v], rows_v, sem)
        _extract_column0(pk_v, fei_v, 16, 0xFF)
        pltpu.sync_copy(fei_v, fei_hbm.at[pl.ds(base, TOK_PER_W)])
        cg.wait()
        pltpu.sync_copy(rows_v, out_hbm.at[pl.ds(base, TOK_PER_W)])

    return _collect


def kernel(hidden_states, W_cls, wi, wo):
    hs = hidden_states.reshape(S, D)
    logits, packed, xs, prow = _router_call(hs, W_cls)
    xbuf, pbuf = _make_dispatch()(xs, prow, packed)
    y = _ffn_call(xbuf, pbuf, wi, wo)
    out, fei = _make_collect()(y, packed)
    return (out.reshape(1, S, D),
            logits.reshape(1, S, E),
            fei.reshape(1, S))
